# Initial kernel scaffold; baseline (speedup 1.0000x reference)
import jax
import jax.numpy as jnp
from jax.experimental import pallas as pl


def kernel(x, edge_attr, emb_W, emb_b, Wm, bm, Wu, bu, ln_g, ln_b, pg_W1, pg_b1, pg_W2, pg_b2, vm_W1, vm_b1, vm_W2, vm_b2, edge_index):
    raise NotImplementedError("write your pallas kernel here")


# speedup vs baseline: 3.3643x; 3.3643x over previous
import functools

import jax
import jax.numpy as jnp
from jax.experimental import pallas as pl
from jax.experimental.pallas import tpu as pltpu

N_BUSES = 256
N_GEN = 32
N_LOADS = 128
H = 256
L = 4
E = 1024
B = 128
PG_MIN = 0.1
PG_MAX = 1.0
VM_MIN = 0.94
VM_MAX = 1.06

BT = 8


def _gnn_kernel(x_ref, ea_ref, embW_ref, embb_ref, Wm_ref, bm_ref, Wu_ref,
                bu_ref, lng_ref, lnb_ref, pgW1_ref, pgb1_ref, pgW2_ref,
                pgb2_ref, vmW1_ref, vmb1_ref, vmW2_ref, vmb2_ref,
                src_ref, dst_ref, pg_out_ref, vm_out_ref):
    f32 = jnp.float32

    src_row = src_ref[...]
    dst_row = dst_ref[...]
    iota_ne = jax.lax.broadcasted_iota(jnp.int32, (N_BUSES, E), 0)
    s_src_t = (iota_ne == src_row).astype(f32)
    s_dst_t = (iota_ne == dst_row).astype(f32)
    deg = jnp.maximum(jnp.sum(s_dst_t, axis=1, keepdims=True), 1.0)

    xb = x_ref[...]
    x_t = xb.T
    pd_t = x_t[:N_LOADS, :]
    qd_t = x_t[N_LOADS:, :]
    zpad_top = jnp.zeros((N_GEN, BT), f32)
    zpad_bot = jnp.zeros((N_BUSES - N_GEN - N_LOADS, BT), f32)
    pd_full = jnp.concatenate([zpad_top, pd_t, zpad_bot], axis=0)
    qd_full = jnp.concatenate([zpad_top, qd_t, zpad_bot], axis=0)

    w = embW_ref[...]
    node_iota = jax.lax.broadcasted_iota(jnp.int32, (N_BUSES, 1), 0)
    gen_mask = (node_iota < N_GEN).astype(f32)
    load_mask = ((node_iota >= N_GEN)
                 & (node_iota < N_GEN + N_LOADS)).astype(f32)
    base = (VM_MIN * w[4:5, :] + VM_MAX * w[5:6, :]
            + gen_mask * (PG_MIN * w[2:3, :] + PG_MAX * w[3:4, :] + w[6:7, :])
            + load_mask * w[7:8, :]
            + embb_ref[...].reshape(1, H))

    h3 = jnp.tanh(pd_full[:, :, None] * w[0:1, :][None, :, :]
                  + qd_full[:, :, None] * w[1:2, :][None, :, :]
                  + base[:, None, :])

    ea = ea_ref[...]
    ea0 = ea[:, 0:1]
    ea1 = ea[:, 1:2]

    dn_gather = (((0,), (0,)), ((), ()))

    for l in range(L):
        wm_l = Wm_ref[l]
        c_l = (ea0 * wm_l[H:H + 1, :] + ea1 * wm_l[H + 1:H + 2, :]
               + bm_ref[l].reshape(1, H))

        h2 = h3.reshape(N_BUSES, BT * H)
        xj2 = jax.lax.dot_general(s_src_t, h2, dn_gather,
                                  preferred_element_type=f32)
        xj_rows = xj2.reshape(E * BT, H)
        m = jnp.dot(xj_rows, wm_l[:H, :],
                    preferred_element_type=f32)
        msg3 = jnp.tanh(m.reshape(E, BT, H) + c_l[:, None, :])
        agg2 = jnp.dot(s_dst_t, msg3.reshape(E, BT * H),
                       preferred_element_type=f32)
        agg3 = agg2.reshape(N_BUSES, BT, H) / deg[:, :, None]

        wu_l = Wu_ref[l]
        h_rows = h3.reshape(N_BUSES * BT, H)
        agg_rows = agg3.reshape(N_BUSES * BT, H)
        upd = (jnp.dot(h_rows, wu_l[:H, :], preferred_element_type=f32)
               + jnp.dot(agg_rows, wu_l[H:, :], preferred_element_type=f32)
               + bu_ref[l].reshape(1, H))
        out = jnp.tanh(upd)

        mu = jnp.mean(out, axis=-1, keepdims=True)
        var = jnp.mean((out - mu) ** 2, axis=-1, keepdims=True)
        normed = ((out - mu) / jnp.sqrt(var + 1e-5)
                  * lng_ref[l].reshape(1, H) + lnb_ref[l].reshape(1, H))
        h3 = (normed + h_rows).reshape(N_BUSES, BT, H)

    hg = h3[:N_GEN].reshape(N_GEN * BT, H)
    t_pg = jnp.tanh(jnp.dot(hg, pgW1_ref[...], preferred_element_type=f32)
                    + pgb1_ref[...].reshape(1, -1))
    pg = (jnp.dot(t_pg, pgW2_ref[...], preferred_element_type=f32)
          + pgb2_ref[...].reshape(1, 1)).reshape(N_GEN, BT)
    t_vm = jnp.tanh(jnp.dot(hg, vmW1_ref[...], preferred_element_type=f32)
                    + vmb1_ref[...].reshape(1, -1))
    vm = (jnp.dot(t_vm, vmW2_ref[...], preferred_element_type=f32)
          + vmb2_ref[...].reshape(1, 1)).reshape(N_GEN, BT)

    pg_out_ref[...] = pg.T
    vm_out_ref[...] = vm.T


@functools.partial(jax.jit, static_argnames=("interpret",))
def _run(x, edge_attr, emb_W, emb_b, Wm, bm, Wu, bu, ln_g, ln_b,
         pg_W1, pg_b1, pg_W2, pg_b2, vm_W1, vm_b1, vm_W2, vm_b2,
         edge_index, interpret=False):
    src_row = edge_index[0].reshape(1, E)
    dst_row = edge_index[1].reshape(1, E)

    grid = (B // BT,)
    full = lambda shape: pl.BlockSpec(shape, lambda i: (0,) * len(shape))
    in_specs = [
        pl.BlockSpec((BT, 2 * N_LOADS), lambda i: (i, 0)),
        full((E, 2)),
        full((9, H)),
        full((H,)),
        full((L, H + 2, H)),
        full((L, H)),
        full((L, 2 * H, H)),
        full((L, H)),
        full((L, H)),
        full((L, H)),
        full((H, H // 2)),
        full((H // 2,)),
        full((H // 2, 1)),
        full((1,)),
        full((H, H // 2)),
        full((H // 2,)),
        full((H // 2, 1)),
        full((1,)),
        full((1, E)),
        full((1, E)),
    ]
    out_specs = [
        pl.BlockSpec((BT, N_GEN), lambda i: (i, 0)),
        pl.BlockSpec((BT, N_GEN), lambda i: (i, 0)),
    ]
    out_shapes = [
        jax.ShapeDtypeStruct((B, N_GEN), jnp.float32),
        jax.ShapeDtypeStruct((B, N_GEN), jnp.float32),
    ]
    pg, vm = pl.pallas_call(
        _gnn_kernel,
        grid=grid,
        in_specs=in_specs,
        out_specs=out_specs,
        out_shape=out_shapes,
        compiler_params=pltpu.CompilerParams(
            dimension_semantics=("arbitrary",)),
        interpret=interpret,
    )(x, edge_attr, emb_W, emb_b, Wm, bm, Wu, bu, ln_g, ln_b,
      pg_W1, pg_b1, pg_W2, pg_b2, vm_W1, vm_b1, vm_W2, vm_b2,
      src_row, dst_row)
    return jnp.concatenate([pg[:, 1:], vm], axis=-1)


def kernel(x, edge_attr, emb_W, emb_b, Wm, bm, Wu, bu, ln_g, ln_b,
           pg_W1, pg_b1, pg_W2, pg_b2, vm_W1, vm_b1, vm_W2, vm_b2,
           edge_index):
    return _run(x, edge_attr, emb_W, emb_b, Wm, bm, Wu, bu, ln_g, ln_b,
                pg_W1, pg_b1, pg_W2, pg_b2, vm_W1, vm_b1, vm_W2, vm_b2,
                edge_index)


# speedup vs baseline: 4.3625x; 1.2967x over previous
import functools

import jax
import jax.numpy as jnp
from jax.experimental import pallas as pl
from jax.experimental.pallas import tpu as pltpu

N_BUSES = 256
N_GEN = 32
N_LOADS = 128
H = 256
L = 4
E = 1024
B = 128
PG_MIN = 0.1
PG_MAX = 1.0
VM_MIN = 0.94
VM_MAX = 1.06

BT = 8


def _gnn_kernel(x_ref, ea_ref, embW_ref, embb_ref, Wm_ref, bm_ref, Wu_ref,
                bu_ref, lng_ref, lnb_ref, pgW1_ref, pgb1_ref, pgW2_ref,
                pgb2_ref, vmW1_ref, vmb1_ref, vmW2_ref, vmb2_ref,
                src_ref, dst_ref, pg_out_ref, vm_out_ref):
    f32 = jnp.float32

    src_row = src_ref[...]
    dst_row = dst_ref[...]
    iota_ne = jax.lax.broadcasted_iota(jnp.int32, (N_BUSES, E), 0)
    s_src_t = (iota_ne == src_row).astype(f32)
    s_dst_t = (iota_ne == dst_row).astype(f32)
    deg = jnp.maximum(jnp.sum(s_dst_t, axis=1, keepdims=True), 1.0)

    xb = x_ref[...]
    x_t = xb.T
    pd_t = x_t[:N_LOADS, :]
    qd_t = x_t[N_LOADS:, :]
    zpad_top = jnp.zeros((N_GEN, BT), f32)
    zpad_bot = jnp.zeros((N_BUSES - N_GEN - N_LOADS, BT), f32)
    pd_full = jnp.concatenate([zpad_top, pd_t, zpad_bot], axis=0)
    qd_full = jnp.concatenate([zpad_top, qd_t, zpad_bot], axis=0)

    w = embW_ref[...]
    node_iota = jax.lax.broadcasted_iota(jnp.int32, (N_BUSES, 1), 0)
    gen_mask = (node_iota < N_GEN).astype(f32)
    load_mask = ((node_iota >= N_GEN)
                 & (node_iota < N_GEN + N_LOADS)).astype(f32)
    base = (VM_MIN * w[4:5, :] + VM_MAX * w[5:6, :]
            + gen_mask * (PG_MIN * w[2:3, :] + PG_MAX * w[3:4, :] + w[6:7, :])
            + load_mask * w[7:8, :]
            + embb_ref[...].reshape(1, H))

    h3 = jnp.tanh(pd_full[:, :, None] * w[0:1, :][None, :, :]
                  + qd_full[:, :, None] * w[1:2, :][None, :, :]
                  + base[:, None, :])

    ea = ea_ref[...]
    ea0 = ea[:, 0:1]
    ea1 = ea[:, 1:2]

    dn_gather = (((0,), (0,)), ((), ()))

    for l in range(L):
        wm_l = Wm_ref[l]
        c_l = (ea0 * wm_l[H:H + 1, :] + ea1 * wm_l[H + 1:H + 2, :]
               + bm_ref[l].reshape(1, H))

        hw = jnp.dot(h3.reshape(N_BUSES * BT, H), wm_l[:H, :],
                     preferred_element_type=f32)
        m2 = jax.lax.dot_general(s_src_t, hw.reshape(N_BUSES, BT * H),
                                 dn_gather,
                                 preferred_element_type=f32)
        msg3 = jnp.tanh(m2.reshape(E, BT, H) + c_l[:, None, :])
        agg2 = jnp.dot(s_dst_t, msg3.reshape(E, BT * H),
                       preferred_element_type=f32)
        agg3 = agg2.reshape(N_BUSES, BT, H) / deg[:, :, None]

        wu_l = Wu_ref[l]
        h_rows = h3.reshape(N_BUSES * BT, H)
        agg_rows = agg3.reshape(N_BUSES * BT, H)
        upd = (jnp.dot(h_rows, wu_l[:H, :], preferred_element_type=f32)
               + jnp.dot(agg_rows, wu_l[H:, :], preferred_element_type=f32)
               + bu_ref[l].reshape(1, H))
        out = jnp.tanh(upd)

        mu = jnp.mean(out, axis=-1, keepdims=True)
        var = jnp.mean((out - mu) ** 2, axis=-1, keepdims=True)
        normed = ((out - mu) / jnp.sqrt(var + 1e-5)
                  * lng_ref[l].reshape(1, H) + lnb_ref[l].reshape(1, H))
        h3 = (normed + h_rows).reshape(N_BUSES, BT, H)

    hg = h3[:N_GEN].reshape(N_GEN * BT, H)
    t_pg = jnp.tanh(jnp.dot(hg, pgW1_ref[...], preferred_element_type=f32)
                    + pgb1_ref[...].reshape(1, -1))
    pg = (jnp.dot(t_pg, pgW2_ref[...], preferred_element_type=f32)
          + pgb2_ref[...].reshape(1, 1)).reshape(N_GEN, BT)
    t_vm = jnp.tanh(jnp.dot(hg, vmW1_ref[...], preferred_element_type=f32)
                    + vmb1_ref[...].reshape(1, -1))
    vm = (jnp.dot(t_vm, vmW2_ref[...], preferred_element_type=f32)
          + vmb2_ref[...].reshape(1, 1)).reshape(N_GEN, BT)

    pg_out_ref[...] = pg.T
    vm_out_ref[...] = vm.T


@functools.partial(jax.jit, static_argnames=("interpret",))
def _run(x, edge_attr, emb_W, emb_b, Wm, bm, Wu, bu, ln_g, ln_b,
         pg_W1, pg_b1, pg_W2, pg_b2, vm_W1, vm_b1, vm_W2, vm_b2,
         edge_index, interpret=False):
    src_row = edge_index[0].reshape(1, E)
    dst_row = edge_index[1].reshape(1, E)

    grid = (B // BT,)
    full = lambda shape: pl.BlockSpec(shape, lambda i: (0,) * len(shape))
    in_specs = [
        pl.BlockSpec((BT, 2 * N_LOADS), lambda i: (i, 0)),
        full((E, 2)),
        full((9, H)),
        full((H,)),
        full((L, H + 2, H)),
        full((L, H)),
        full((L, 2 * H, H)),
        full((L, H)),
        full((L, H)),
        full((L, H)),
        full((H, H // 2)),
        full((H // 2,)),
        full((H // 2, 1)),
        full((1,)),
        full((H, H // 2)),
        full((H // 2,)),
        full((H // 2, 1)),
        full((1,)),
        full((1, E)),
        full((1, E)),
    ]
    out_specs = [
        pl.BlockSpec((BT, N_GEN), lambda i: (i, 0)),
        pl.BlockSpec((BT, N_GEN), lambda i: (i, 0)),
    ]
    out_shapes = [
        jax.ShapeDtypeStruct((B, N_GEN), jnp.float32),
        jax.ShapeDtypeStruct((B, N_GEN), jnp.float32),
    ]
    pg, vm = pl.pallas_call(
        _gnn_kernel,
        grid=grid,
        in_specs=in_specs,
        out_specs=out_specs,
        out_shape=out_shapes,
        compiler_params=pltpu.CompilerParams(
            dimension_semantics=("parallel",)),
        interpret=interpret,
    )(x, edge_attr, emb_W, emb_b, Wm, bm, Wu, bu, ln_g, ln_b,
      pg_W1, pg_b1, pg_W2, pg_b2, vm_W1, vm_b1, vm_W2, vm_b2,
      src_row, dst_row)
    return jnp.concatenate([pg[:, 1:], vm], axis=-1)


def kernel(x, edge_attr, emb_W, emb_b, Wm, bm, Wu, bu, ln_g, ln_b,
           pg_W1, pg_b1, pg_W2, pg_b2, vm_W1, vm_b1, vm_W2, vm_b2,
           edge_index):
    return _run(x, edge_attr, emb_W, emb_b, Wm, bm, Wu, bu, ln_g, ln_b,
                pg_W1, pg_b1, pg_W2, pg_b2, vm_W1, vm_b1, vm_W2, vm_b2,
                edge_index)
